# baseline (speedup 1.0000x reference)
import functools

import jax
import jax.numpy as jnp
from jax.experimental import pallas as pl


def _body(x_ref, lo_ref, invw_ref, w_ref, b_ref, g_ref, bb_ref, o_ref, *, F, NB, D):
    xblk = x_ref[...]
    g = g_ref[...]
    bb = bb_ref[...]
    for f in range(F):
        v = xblk[:, f:f + 1]
        lo_f = lo_ref[f:f + 1, :]
        invw_f = invw_ref[f:f + 1, :]
        enc = jnp.clip((v - lo_f) * invw_f, 0.0, 1.0)
        emb = jnp.dot(enc, w_ref[f], preferred_element_type=jnp.float32)
        emb = emb + b_ref[f:f + 1, :]
        mean = jnp.mean(emb, axis=-1, keepdims=True)
        zc = emb - mean
        var = jnp.mean(zc * zc, axis=-1, keepdims=True)
        o = zc * jax.lax.rsqrt(var + 1e-5) * g + bb
        o_ref[:, f, :] = o


def kernel(x, bin_edges, W, b, ln_gamma, ln_beta):
    B, O, F = x.shape
    NB = bin_edges.shape[1] - 1
    D = W.shape[2]
    N = B * O
    RB = 256
    x2 = x.reshape(N, F)
    lo = bin_edges[:, :-1]
    invw = 1.0 / (bin_edges[:, 1:] - bin_edges[:, :-1])
    g2 = ln_gamma.reshape(1, D)
    bb2 = ln_beta.reshape(1, D)

    out = pl.pallas_call(
        functools.partial(_body, F=F, NB=NB, D=D),
        grid=(N // RB,),
        in_specs=[
            pl.BlockSpec((RB, F), lambda i: (i, 0)),
            pl.BlockSpec((F, NB), lambda i: (0, 0)),
            pl.BlockSpec((F, NB), lambda i: (0, 0)),
            pl.BlockSpec((F, NB, D), lambda i: (0, 0, 0)),
            pl.BlockSpec((F, D), lambda i: (0, 0)),
            pl.BlockSpec((1, D), lambda i: (0, 0)),
            pl.BlockSpec((1, D), lambda i: (0, 0)),
        ],
        out_specs=pl.BlockSpec((RB, F, D), lambda i: (i, 0, 0)),
        out_shape=jax.ShapeDtypeStruct((N, F, D), jnp.float32),
    )(x2, lo, invw, W, b, g2, bb2)
    return out.reshape(B, O, F, D)
